# baseline (speedup 1.0000x reference)
import jax
import jax.numpy as jnp
from jax.experimental import pallas as pl
from jax.experimental.pallas import tpu as pltpu

_H = 32
_W = 32
_HW = _H * _W
_CP = 640
_CS = 160
_CSP = 256
_PAD = _W + 1
_ROWS = _HW + 2 * _PAD
_HM = 2
_EMB = 512
_NS = 128
_KN = 9 * _CSP


def _bn_lrelu(acc, s_ref, b_ref):
    y = acc * s_ref[0] + b_ref[0]
    return jnp.where(y > 0.0, y, 0.01 * y)


def _fused_kernel(cols_ref, ws_ref, ss_ref, bs_ref, w1_ref, s1_ref, b1_ref,
                  w2a_ref, w2b_ref, w2c_ref, w2d_ref, s2_ref, b2_ref,
                  we_ref, se_ref, be_ref, wh_ref, sh_ref, bh_ref,
                  oe_ref, oh_ref, *scr):
    f32 = jnp.float32
    bf = jnp.bfloat16

    sbufs = scr[0:3]
    abufs = [scr[3 + 3 * b:6 + 3 * b] for b in range(4)]
    cbufs = [scr[15 + 3 * b:18 + 3 * b] for b in range(4)]
    col = scr[27]

    for buf in scr[:27]:
        w = buf.shape[1]
        buf[0:_PAD, :] = jnp.zeros((_PAD, w), bf)
        buf[_ROWS - _PAD - 2:_ROWS, :] = jnp.zeros((_PAD + 2, w), bf)

    cidx = jax.lax.broadcasted_iota(jnp.int32, (_HW, 1), 0) % _W
    not_last = cidx != (_W - 1)
    not_first = cidx != 0
    zero = jnp.zeros((), bf)

    def store3(y, bufs):
        yb = y.astype(bf)
        if yb.shape[1] < bufs[0].shape[1]:
            yb = jnp.concatenate(
                [yb, jnp.zeros((_HW, bufs[0].shape[1] - yb.shape[1]), bf)],
                axis=1)
        bufs[0][_PAD:_PAD + _HW, :] = jnp.where(not_last, yb, zero)
        bufs[1][_PAD - 1:_PAD - 1 + _HW, :] = yb
        bufs[2][_PAD - 2:_PAD - 2 + _HW, :] = jnp.where(not_first, yb, zero)

    def build_col(bufs, slot0):
        for t in range(9):
            dy, dx = t // 3, t % 3
            col[:, (slot0 + t) * _CSP:(slot0 + t + 1) * _CSP] = (
                bufs[dx][dy * _W:dy * _W + _HW, :])

    y = jnp.dot(cols_ref[0], ws_ref[...], preferred_element_type=f32)
    store3(_bn_lrelu(y, ss_ref, bs_ref), sbufs)

    build_col(sbufs, 0)
    y = jnp.dot(col[:, :_KN], w1_ref[...], preferred_element_type=f32)
    y = _bn_lrelu(y, s1_ref, b1_ref)
    for b in range(4):
        store3(y[:, b * _CS:(b + 1) * _CS], abufs[b])

    w2 = (w2a_ref, w2b_ref, w2c_ref, w2d_ref)
    for b in range(4):
        build_col(abufs[b], 0)
        yb = jnp.dot(col[:, :_KN], w2[b][...], preferred_element_type=f32)
        yb = yb * s2_ref[0, b * _CSP:(b + 1) * _CSP] + \
            b2_ref[0, b * _CSP:(b + 1) * _CSP]
        yb = jnp.where(yb > 0.0, yb, 0.01 * yb)
        store3(yb, cbufs[b])

    for b in range(4):
        build_col(cbufs[b], 9 * b)
    ye = jnp.dot(col[:, 3 * _KN:4 * _KN], we_ref[...],
                 preferred_element_type=f32)
    ye = _bn_lrelu(ye, se_ref, be_ref)
    yh = jnp.dot(col[:, :3 * _KN], wh_ref[...], preferred_element_type=f32)
    yh = _bn_lrelu(yh, sh_ref, bh_ref)

    ch = jax.lax.broadcasted_iota(jnp.int32, (_HW, _NS), 1)
    is_hm = ch < _HM
    zhm = jnp.where(is_hm, yh, -jnp.inf)
    m = jnp.max(zhm, axis=-1, keepdims=True)
    e = jnp.exp(zhm - m)
    hm = e / jnp.sum(e, axis=-1, keepdims=True)
    oh_ref[...] = jnp.where(is_hm, hm, yh).reshape(1, _HW, _NS)

    nrm = jnp.maximum(
        jnp.sqrt(jnp.sum(ye * ye, axis=-1, keepdims=True)), 1e-12)
    oe_ref[...] = (ye / nrm).reshape(1, _HW, _EMB)


def _stem_cols(x_nhwc):
    N, H, W, Cimg = x_nhwc.shape
    xp = jnp.pad(x_nhwc, ((0, 0), (1, 1), (1, 1), (0, 0)))
    taps = [xp[:, dy:dy + H, dx:dx + W, :]
            for dy in range(3) for dx in range(3)]
    cols = jnp.concatenate(taps, axis=-1).reshape(N, H * W, 9 * Cimg)
    kpad = (-(9 * Cimg)) % 8
    if kpad:
        cols = jnp.pad(cols, ((0, 0), (0, 0), (0, kpad)))
    return cols.astype(jnp.bfloat16)


def kernel(img, ws, ss, bs, w1, s1, b1, w2, s2, b2, w3, s3, b3):
    N = img.shape[0]
    x = jnp.transpose(img, (0, 2, 3, 1)).astype(jnp.float32)
    cols = _stem_cols(x)
    KS = cols.shape[-1]
    bf = jnp.bfloat16
    emb_off = _HM + 4

    ws_n = ws[:, :_CS]
    ss_n = ss[:, :_CS]
    bs_n = bs[:, :_CS]
    w1_n = jnp.zeros((9, _CSP, _CP), bf).at[:, :_CS, :].set(
        w1.reshape(9, _CP, _CP)[:, :_CS, :].astype(bf)).reshape(_KN, _CP)

    w2r = w2.reshape(9, _CP, _CP)
    w2bl = []
    for b in range(4):
        lo = b * _CS
        blk = jnp.zeros((9, _CSP, _CSP), bf).at[:, :_CS, :_CS].set(
            w2r[:, lo:lo + _CS, lo:lo + _CS].astype(bf))
        w2bl.append(blk.reshape(_KN, _CSP))
    s2p = jnp.ones((1, 4 * _CSP), s2.dtype)
    b2p = jnp.zeros((1, 4 * _CSP), b2.dtype)
    for b in range(4):
        s2p = s2p.at[:, b * _CSP:b * _CSP + _CS].set(
            s2[:, b * _CS:(b + 1) * _CS])
        b2p = b2p.at[:, b * _CSP:b * _CSP + _CS].set(
            b2[:, b * _CS:(b + 1) * _CS])

    w3r = w3.reshape(9, _CP, _CP)
    w_e = jnp.zeros((9, _CSP, _EMB), bf).at[:, :_CS, :].set(
        w3r[:, 3 * _CS:_CP, emb_off:emb_off + _EMB].astype(bf)).reshape(
            _KN, _EMB)
    s_e = s3[:, emb_off:emb_off + _EMB]
    b_e = b3[:, emb_off:emb_off + _EMB]
    w_h = jnp.zeros((3, 9, _CSP, _NS), bf)
    for b in range(3):
        w_h = w_h.at[b, :, :_CS, :6].set(
            w3r[:, b * _CS:(b + 1) * _CS, :6].astype(bf))
    w_h = w_h.reshape(3 * _KN, _NS)
    s_h = jnp.ones((1, _NS), s3.dtype).at[:, :6].set(s3[:, :6])
    b_h = jnp.zeros((1, _NS), b3.dtype).at[:, :6].set(b3[:, :6])

    wspec = lambda shape: pl.BlockSpec(shape, lambda n: (0, 0))
    vm = lambda w: pltpu.VMEM((_ROWS, w), bf)
    oe, oh = pl.pallas_call(
        _fused_kernel,
        out_shape=(jax.ShapeDtypeStruct((N, _HW, _EMB), jnp.float32),
                   jax.ShapeDtypeStruct((N, _HW, _NS), jnp.float32)),
        grid=(N,),
        in_specs=[
            pl.BlockSpec((1, _HW, KS), lambda n: (n, 0, 0)),
            wspec((KS, _CS)), wspec((1, _CS)), wspec((1, _CS)),
            wspec((_KN, _CP)), wspec((1, _CP)), wspec((1, _CP)),
            wspec((_KN, _CSP)), wspec((_KN, _CSP)),
            wspec((_KN, _CSP)), wspec((_KN, _CSP)),
            wspec((1, 4 * _CSP)), wspec((1, 4 * _CSP)),
            wspec((_KN, _EMB)), wspec((1, _EMB)), wspec((1, _EMB)),
            wspec((3 * _KN, _NS)), wspec((1, _NS)), wspec((1, _NS)),
        ],
        out_specs=(pl.BlockSpec((1, _HW, _EMB), lambda n: (n, 0, 0)),
                   pl.BlockSpec((1, _HW, _NS), lambda n: (n, 0, 0))),
        scratch_shapes=(
            [vm(_CSP)] * 3 + [vm(_CSP)] * 12 + [vm(_CSP)] * 12
            + [pltpu.VMEM((_HW, 4 * _KN), bf)]),
        compiler_params=pltpu.CompilerParams(
            dimension_semantics=("parallel",)),
    )(cols, ws_n, ss_n, bs_n, w1_n, s1, b1,
      w2bl[0], w2bl[1], w2bl[2], w2bl[3], s2p, b2p,
      w_e, s_e, b_e, w_h, s_h, b_h)

    oh = oh.reshape(N, _H, _W, _NS)
    heatmap = oh[..., 0:_HM]
    offset = oh[..., _HM:_HM + 2]
    wh = oh[..., _HM + 2:_HM + 4]
    emb = oe.reshape(N, _H, _W, _EMB)
    to_nchw = lambda t: jnp.transpose(t, (0, 3, 1, 2))
    return to_nchw(heatmap), to_nchw(offset), to_nchw(wh), to_nchw(emb)


# speedup vs baseline: 1.0118x; 1.0118x over previous
import jax
import jax.numpy as jnp
from jax.experimental import pallas as pl
from jax.experimental.pallas import tpu as pltpu

_H = 32
_W = 32
_HW = _H * _W
_CP = 640
_CS = 160
_CSP = 256
_PAD = _W + 1
_ROWS = _HW + 2 * _PAD
_HM = 2
_EMB = 512
_NS = 128
_KN = 9 * _CSP
_KW = 9 * _CP


def _bn_lrelu(acc, s_ref, b_ref):
    y = acc * s_ref[0] + b_ref[0]
    return jnp.where(y > 0.0, y, 0.01 * y)


def _fused_kernel(cols_ref, ws_ref, ss_ref, bs_ref,
                  w1_ref, s1_ref, b1_ref,
                  w2_ref, s2_ref, b2_ref,
                  we_ref, se_ref, be_ref,
                  wh_ref, sh_ref, bh_ref,
                  oe_ref, oh_ref,
                  s0, s1b, s2b, a0, a1, a2, c0, c1, c2, col):
    f32 = jnp.float32
    bf = jnp.bfloat16

    for b0, b1, b2 in ((s0, s1b, s2b), (a0, a1, a2), (c0, c1, c2)):
        for buf in (b0, b1, b2):
            w = buf.shape[1]
            buf[0:_PAD, :] = jnp.zeros((_PAD, w), bf)
            buf[_ROWS - _PAD - 2:_ROWS, :] = jnp.zeros((_PAD + 2, w), bf)

    cidx = jax.lax.broadcasted_iota(jnp.int32, (_HW, 1), 0) % _W
    not_last = cidx != (_W - 1)
    not_first = cidx != 0
    zero = jnp.zeros((), bf)

    def store3(y, b0, b1, b2, padto=None):
        yb = y.astype(bf)
        if padto is not None:
            yb = jnp.concatenate(
                [yb, jnp.zeros((_HW, padto - yb.shape[1]), bf)], axis=1)
        b0[_PAD:_PAD + _HW, :] = jnp.where(not_last, yb, zero)
        b1[_PAD - 1:_PAD - 1 + _HW, :] = yb
        b2[_PAD - 2:_PAD - 2 + _HW, :] = jnp.where(not_first, yb, zero)

    def build_col(b0, b1, b2, w):
        bufs = (b0, b1, b2)
        for t in range(9):
            dy, dx = t // 3, t % 3
            col[:, t * w:(t + 1) * w] = bufs[dx][dy * _W:dy * _W + _HW, :]

    y = jnp.dot(cols_ref[0], ws_ref[...], preferred_element_type=f32)
    store3(_bn_lrelu(y, ss_ref, bs_ref), s0, s1b, s2b, padto=_CSP)

    build_col(s0, s1b, s2b, _CSP)
    y = jnp.dot(col[:, :_KN], w1_ref[...], preferred_element_type=f32)
    store3(_bn_lrelu(y, s1_ref, b1_ref), a0, a1, a2)

    build_col(a0, a1, a2, _CP)
    y = jnp.dot(col[...], w2_ref[...], preferred_element_type=f32)
    y = _bn_lrelu(y, s2_ref, b2_ref)
    store3(y, c0, c1, c2)

    cbufs = (c0, c1, c2)
    for t in range(9):
        dy, dx = t // 3, t % 3
        col[:, t * _CSP:(t + 1) * _CSP] = (
            cbufs[dx][dy * _W:dy * _W + _HW, :_CSP])
    ye = jnp.dot(col[:, :_KN], we_ref[...], preferred_element_type=f32)
    ye = _bn_lrelu(ye, se_ref, be_ref)

    acc = None
    for t in range(9):
        dy, dx = t // 3, t % 3
        slab = cbufs[dx][dy * _W:dy * _W + _HW, :]
        d = jnp.dot(slab, wh_ref[t * _CP:(t + 1) * _CP, :],
                    preferred_element_type=f32)
        acc = d if acc is None else acc + d
    yh = _bn_lrelu(acc, sh_ref, bh_ref)

    ch = jax.lax.broadcasted_iota(jnp.int32, (_HW, _NS), 1)
    is_hm = ch < _HM
    zhm = jnp.where(is_hm, yh, -jnp.inf)
    m = jnp.max(zhm, axis=-1, keepdims=True)
    e = jnp.exp(zhm - m)
    hm = e / jnp.sum(e, axis=-1, keepdims=True)
    oh_ref[...] = jnp.where(is_hm, hm, yh).reshape(1, _HW, _NS)

    nrm = jnp.maximum(
        jnp.sqrt(jnp.sum(ye * ye, axis=-1, keepdims=True)), 1e-12)
    oe_ref[...] = (ye / nrm).reshape(1, _HW, _EMB)


def _stem_cols(x_nhwc):
    N, H, W, Cimg = x_nhwc.shape
    xp = jnp.pad(x_nhwc, ((0, 0), (1, 1), (1, 1), (0, 0)))
    taps = [xp[:, dy:dy + H, dx:dx + W, :]
            for dy in range(3) for dx in range(3)]
    cols = jnp.concatenate(taps, axis=-1).reshape(N, H * W, 9 * Cimg)
    kpad = (-(9 * Cimg)) % 8
    if kpad:
        cols = jnp.pad(cols, ((0, 0), (0, 0), (0, kpad)))
    return cols.astype(jnp.bfloat16)


def kernel(img, ws, ss, bs, w1, s1, b1, w2, s2, b2, w3, s3, b3):
    N = img.shape[0]
    x = jnp.transpose(img, (0, 2, 3, 1)).astype(jnp.float32)
    cols = _stem_cols(x)
    KS = cols.shape[-1]
    bf = jnp.bfloat16
    emb_off = _HM + 4

    ws_n = ws[:, :_CS]
    ss_n = ss[:, :_CS]
    bs_n = bs[:, :_CS]
    w1_n = jnp.zeros((9, _CSP, _CP), bf).at[:, :_CS, :].set(
        w1.reshape(9, _CP, _CP)[:, :_CS, :].astype(bf)).reshape(_KN, _CP)

    perm = jnp.concatenate([jnp.arange(3 * _CS, _CP), jnp.arange(3 * _CS)])
    w2p = w2.reshape(9, _CP, _CP)[:, :, perm].reshape(_KW, _CP)
    s2p = s2[:, perm]
    b2p = b2[:, perm]
    w3p = w3.reshape(9, _CP, _CP)[:, perm, :]

    w_e = jnp.zeros((9, _CSP, _EMB), bf).at[:, :_CS, :].set(
        w3p[:, :_CS, emb_off:emb_off + _EMB].astype(bf)).reshape(_KN, _EMB)
    s_e = s3[:, emb_off:emb_off + _EMB]
    b_e = b3[:, emb_off:emb_off + _EMB]
    w_h = jnp.zeros((9, _CP, _NS), bf).at[:, :, :6].set(
        w3p[:, :, :6].astype(bf)).reshape(_KW, _NS)
    s_h = jnp.ones((1, _NS), s3.dtype).at[:, :6].set(s3[:, :6])
    b_h = jnp.zeros((1, _NS), b3.dtype).at[:, :6].set(b3[:, :6])

    wspec = lambda shape: pl.BlockSpec(shape, lambda n: (0, 0))
    vm = lambda w: pltpu.VMEM((_ROWS, w), bf)
    oe, oh = pl.pallas_call(
        _fused_kernel,
        out_shape=(jax.ShapeDtypeStruct((N, _HW, _EMB), jnp.float32),
                   jax.ShapeDtypeStruct((N, _HW, _NS), jnp.float32)),
        grid=(N,),
        in_specs=[
            pl.BlockSpec((1, _HW, KS), lambda n: (n, 0, 0)),
            wspec((KS, _CS)), wspec((1, _CS)), wspec((1, _CS)),
            wspec((_KN, _CP)), wspec((1, _CP)), wspec((1, _CP)),
            wspec((_KW, _CP)), wspec((1, _CP)), wspec((1, _CP)),
            wspec((_KN, _EMB)), wspec((1, _EMB)), wspec((1, _EMB)),
            wspec((_KW, _NS)), wspec((1, _NS)), wspec((1, _NS)),
        ],
        out_specs=(pl.BlockSpec((1, _HW, _EMB), lambda n: (n, 0, 0)),
                   pl.BlockSpec((1, _HW, _NS), lambda n: (n, 0, 0))),
        scratch_shapes=[
            vm(_CSP), vm(_CSP), vm(_CSP),
            vm(_CP), vm(_CP), vm(_CP),
            vm(_CP), vm(_CP), vm(_CP),
            pltpu.VMEM((_HW, _KW), bf),
        ],
        compiler_params=pltpu.CompilerParams(
            dimension_semantics=("parallel",)),
    )(cols, ws_n, ss_n, bs_n,
      w1_n, s1, b1, w2p.astype(bf), s2p, b2p,
      w_e, s_e, b_e, w_h, s_h, b_h)

    oh = oh.reshape(N, _H, _W, _NS)
    heatmap = oh[..., 0:_HM]
    offset = oh[..., _HM:_HM + 2]
    wh = oh[..., _HM + 2:_HM + 4]
    emb = oe.reshape(N, _H, _W, _EMB)
    to_nchw = lambda t: jnp.transpose(t, (0, 3, 1, 2))
    return to_nchw(heatmap), to_nchw(offset), to_nchw(wh), to_nchw(emb)


# speedup vs baseline: 1.0918x; 1.0791x over previous
import jax
import jax.numpy as jnp
from jax.experimental import pallas as pl
from jax.experimental.pallas import tpu as pltpu

_H = 32
_W = 32
_HW = _H * _W
_CP = 640
_CS = 160
_CSP = 256
_CG = 320
_CGP = 384
_PAD = _W + 1
_ROWS = _HW + 2 * _PAD
_HM = 2
_EMB = 512
_NS = 128
_KN = 9 * _CSP
_KG = 9 * _CGP


def _bn_lrelu(acc, s_ref, b_ref):
    y = acc * s_ref[0] + b_ref[0]
    return jnp.where(y > 0.0, y, 0.01 * y)


def _fused_kernel(cols_ref, ws_ref, ss_ref, bs_ref,
                  w1_ref, s1_ref, b1_ref,
                  w2a_ref, s2a_ref, b2a_ref,
                  w2b_ref, s2b_ref, b2b_ref,
                  we_ref, se_ref, be_ref,
                  wh0_ref, wh1_ref, sh_ref, bh_ref,
                  oe_ref, oh_ref,
                  s0, s1b, s2b, a00, a01, a02, a10, a11, a12,
                  c00, c01, c02, c10, c11, c12, col):
    f32 = jnp.float32
    bf = jnp.bfloat16

    sbufs = (s0, s1b, s2b)
    abufs = ((a00, a01, a02), (a10, a11, a12))
    cbufs = ((c00, c01, c02), (c10, c11, c12))

    for bufs in (sbufs,) + abufs + cbufs:
        for buf in bufs:
            w = buf.shape[1]
            buf[0:_PAD, :] = jnp.zeros((_PAD, w), bf)
            buf[_ROWS - _PAD - 2:_ROWS, :] = jnp.zeros((_PAD + 2, w), bf)

    cidx = jax.lax.broadcasted_iota(jnp.int32, (_HW, 1), 0) % _W
    not_last = cidx != (_W - 1)
    not_first = cidx != 0
    zero = jnp.zeros((), bf)

    def store3(y, bufs):
        yb = y.astype(bf)
        if yb.shape[1] < bufs[0].shape[1]:
            yb = jnp.concatenate(
                [yb, jnp.zeros((_HW, bufs[0].shape[1] - yb.shape[1]), bf)],
                axis=1)
        bufs[0][_PAD:_PAD + _HW, :] = jnp.where(not_last, yb, zero)
        bufs[1][_PAD - 1:_PAD - 1 + _HW, :] = yb
        bufs[2][_PAD - 2:_PAD - 2 + _HW, :] = jnp.where(not_first, yb, zero)

    def build_col(bufs, w, lanes=None):
        for t in range(9):
            dy, dx = t // 3, t % 3
            src = bufs[dx][dy * _W:dy * _W + _HW, :]
            if lanes is not None:
                src = src[:, :lanes]
            col[:, t * w:(t + 1) * w] = src

    y = jnp.dot(cols_ref[0], ws_ref[...], preferred_element_type=f32)
    store3(_bn_lrelu(y, ss_ref, bs_ref), sbufs)

    build_col(sbufs, _CSP)
    y = jnp.dot(col[:, :_KN], w1_ref[...], preferred_element_type=f32)
    y = _bn_lrelu(y, s1_ref, b1_ref)
    store3(y[:, :_CG], abufs[0])
    store3(y[:, _CG:_CP], abufs[1])

    for g, w2g, s2g, b2g in ((0, w2a_ref, s2a_ref, b2a_ref),
                             (1, w2b_ref, s2b_ref, b2b_ref)):
        build_col(abufs[g], _CGP)
        yg = jnp.dot(col[:, :_KG], w2g[...], preferred_element_type=f32)
        store3(_bn_lrelu(yg, s2g, b2g), cbufs[g])

    build_col(cbufs[1], _CSP, lanes=_CSP)
    ye = jnp.dot(col[:, :_KN], we_ref[...], preferred_element_type=f32)
    ye = _bn_lrelu(ye, se_ref, be_ref)

    acc = None
    for g, whg in ((0, wh0_ref), (1, wh1_ref)):
        for t in range(9):
            dy, dx = t // 3, t % 3
            slab = cbufs[g][dx][dy * _W:dy * _W + _HW, :]
            d = jnp.dot(slab, whg[t * _CGP:(t + 1) * _CGP, :],
                        preferred_element_type=f32)
            acc = d if acc is None else acc + d
    yh = _bn_lrelu(acc, sh_ref, bh_ref)

    ch = jax.lax.broadcasted_iota(jnp.int32, (_HW, _NS), 1)
    is_hm = ch < _HM
    zhm = jnp.where(is_hm, yh, -jnp.inf)
    m = jnp.max(zhm, axis=-1, keepdims=True)
    e = jnp.exp(zhm - m)
    hm = e / jnp.sum(e, axis=-1, keepdims=True)
    oh_ref[...] = jnp.where(is_hm, hm, yh).reshape(1, _HW, _NS)

    nrm = jnp.maximum(
        jnp.sqrt(jnp.sum(ye * ye, axis=-1, keepdims=True)), 1e-12)
    oe_ref[...] = (ye / nrm).reshape(1, _HW, _EMB)


def _stem_cols(x_nhwc):
    N, H, W, Cimg = x_nhwc.shape
    xp = jnp.pad(x_nhwc, ((0, 0), (1, 1), (1, 1), (0, 0)))
    taps = [xp[:, dy:dy + H, dx:dx + W, :]
            for dy in range(3) for dx in range(3)]
    cols = jnp.concatenate(taps, axis=-1).reshape(N, H * W, 9 * Cimg)
    kpad = (-(9 * Cimg)) % 8
    if kpad:
        cols = jnp.pad(cols, ((0, 0), (0, 0), (0, kpad)))
    return cols.astype(jnp.bfloat16)


def kernel(img, ws, ss, bs, w1, s1, b1, w2, s2, b2, w3, s3, b3):
    N = img.shape[0]
    x = jnp.transpose(img, (0, 2, 3, 1)).astype(jnp.float32)
    cols = _stem_cols(x)
    KS = cols.shape[-1]
    bf = jnp.bfloat16
    emb_off = _HM + 4

    ws_n = ws[:, :_CS]
    ss_n = ss[:, :_CS]
    bs_n = bs[:, :_CS]
    w1_n = jnp.zeros((9, _CSP, _CP), bf).at[:, :_CS, :].set(
        w1.reshape(9, _CP, _CP)[:, :_CS, :].astype(bf)).reshape(_KN, _CP)

    w2r = w2.reshape(9, _CP, _CP)
    g1cols = jnp.concatenate([jnp.arange(3 * _CS, _CP),
                              jnp.arange(2 * _CS, 3 * _CS)])
    w2a = jnp.zeros((9, _CGP, _CGP), bf).at[:, :_CG, :_CG].set(
        w2r[:, :_CG, :_CG].astype(bf)).reshape(_KG, _CGP)
    w2b = jnp.zeros((9, _CGP, _CGP), bf).at[:, :_CG, :_CG].set(
        w2r[:, _CG:_CP, g1cols].astype(bf)).reshape(_KG, _CGP)
    padv = lambda v, fill: jnp.full((1, _CGP), fill, v.dtype).at[
        :, :_CG].set(v)
    s2a = padv(s2[:, :_CG], 1.0)
    b2a = padv(b2[:, :_CG], 0.0)
    s2g1 = jnp.concatenate([s2[:, 3 * _CS:], s2[:, 2 * _CS:3 * _CS]], axis=1)
    b2g1 = jnp.concatenate([b2[:, 3 * _CS:], b2[:, 2 * _CS:3 * _CS]], axis=1)
    s2bp = padv(s2g1, 1.0)
    b2bp = padv(b2g1, 0.0)

    w3r = w3.reshape(9, _CP, _CP)
    w_e = jnp.zeros((9, _CSP, _EMB), bf).at[:, :_CS, :].set(
        w3r[:, 3 * _CS:_CP, emb_off:emb_off + _EMB].astype(bf)).reshape(
            _KN, _EMB)
    s_e = s3[:, emb_off:emb_off + _EMB]
    b_e = b3[:, emb_off:emb_off + _EMB]
    w_h0 = jnp.zeros((9, _CGP, _NS), bf).at[:, :_CG, :6].set(
        w3r[:, :_CG, :6].astype(bf)).reshape(_KG, _NS)
    w_h1 = jnp.zeros((9, _CGP, _NS), bf).at[:, _CS:_CG, :6].set(
        w3r[:, 2 * _CS:3 * _CS, :6].astype(bf)).reshape(_KG, _NS)
    s_h = jnp.ones((1, _NS), s3.dtype).at[:, :6].set(s3[:, :6])
    b_h = jnp.zeros((1, _NS), b3.dtype).at[:, :6].set(b3[:, :6])

    wspec = lambda shape: pl.BlockSpec(shape, lambda n: (0, 0))
    vm = lambda w: pltpu.VMEM((_ROWS, w), bf)
    oe, oh = pl.pallas_call(
        _fused_kernel,
        out_shape=(jax.ShapeDtypeStruct((N, _HW, _EMB), jnp.float32),
                   jax.ShapeDtypeStruct((N, _HW, _NS), jnp.float32)),
        grid=(N,),
        in_specs=[
            pl.BlockSpec((1, _HW, KS), lambda n: (n, 0, 0)),
            wspec((KS, _CS)), wspec((1, _CS)), wspec((1, _CS)),
            wspec((_KN, _CP)), wspec((1, _CP)), wspec((1, _CP)),
            wspec((_KG, _CGP)), wspec((1, _CGP)), wspec((1, _CGP)),
            wspec((_KG, _CGP)), wspec((1, _CGP)), wspec((1, _CGP)),
            wspec((_KN, _EMB)), wspec((1, _EMB)), wspec((1, _EMB)),
            wspec((_KG, _NS)), wspec((_KG, _NS)),
            wspec((1, _NS)), wspec((1, _NS)),
        ],
        out_specs=(pl.BlockSpec((1, _HW, _EMB), lambda n: (n, 0, 0)),
                   pl.BlockSpec((1, _HW, _NS), lambda n: (n, 0, 0))),
        scratch_shapes=(
            [vm(_CSP)] * 3 + [vm(_CGP)] * 6 + [vm(_CGP)] * 6
            + [pltpu.VMEM((_HW, _KG), bf)]),
        compiler_params=pltpu.CompilerParams(
            dimension_semantics=("parallel",)),
    )(cols, ws_n, ss_n, bs_n, w1_n, s1, b1,
      w2a, s2a, b2a, w2b, s2bp, b2bp,
      w_e, s_e, b_e, w_h0, w_h1, s_h, b_h)

    oh = oh.reshape(N, _H, _W, _NS)
    heatmap = oh[..., 0:_HM]
    offset = oh[..., _HM:_HM + 2]
    wh = oh[..., _HM + 2:_HM + 4]
    emb = oe.reshape(N, _H, _W, _EMB)
    to_nchw = lambda t: jnp.transpose(t, (0, 3, 1, 2))
    return to_nchw(heatmap), to_nchw(offset), to_nchw(wh), to_nchw(emb)
